# Initial kernel scaffold; baseline (speedup 1.0000x reference)
import jax
import jax.numpy as jnp
from jax.experimental import pallas as pl


def kernel(features, edge_index, W1, b1, W2, b2, W3, b3, Wf, bf):
    raise NotImplementedError("write your pallas kernel here")


# speedup vs baseline: 5.6145x; 5.6145x over previous
import functools

import jax
import jax.numpy as jnp
from jax import lax
from jax.experimental import pallas as pl
from jax.experimental.pallas import tpu as pltpu
from jax.experimental.pallas import tpu_sc as plsc

N = 10000
E = 320000
D = 128
DOUT = 64

NC = 2
NS = 16
LW = 128
RPT = 80
EP_ROWS = NC * NS * RPT
EP = EP_ROWS * LW
NH = 10240
NPS = NH // NS
NPAD = NH - N

_mesh = plsc.VectorSubcoreMesh(
    core_axis_name="c", subcore_axis_name="s", num_cores=NC, num_subcores=NS
)


@functools.partial(
    pl.kernel,
    out_type=jax.ShapeDtypeStruct((NC, NH, D), jnp.float32),
    mesh=_mesh,
    scratch_types=[
        pltpu.VMEM((RPT, LW), jnp.int32),
        pltpu.VMEM((RPT, LW), jnp.int32),
        pltpu.VMEM((LW, D), jnp.float32),
        pltpu.VMEM_SHARED((NH, D), jnp.float32),
        pltpu.SemaphoreType.DMA,
    ],
)
def _sc_aggregate(h_hbm, src2d, dst2d, zeros_hbm, out_hbm,
                  idx_s, idx_d, rows_v, acc, sem):
    c = lax.axis_index("c")
    s = lax.axis_index("s")
    pltpu.sync_copy(zeros_hbm, acc.at[pl.ds(s * NPS, NPS)])
    base = (c * NS + s) * RPT
    pltpu.sync_copy(src2d.at[pl.ds(base, RPT)], idx_s)
    pltpu.sync_copy(dst2d.at[pl.ds(base, RPT)], idx_d)
    plsc.subcore_barrier()

    def body(i, carry):
        pltpu.async_copy(h_hbm.at[idx_s.at[i]], rows_v, sem).wait()
        pltpu.sync_copy(rows_v, acc.at[idx_d.at[i]], add=True)
        return carry

    lax.fori_loop(0, RPT, body, 0)
    plsc.subcore_barrier()
    pltpu.sync_copy(acc.at[pl.ds(s * NPS, NPS)],
                    out_hbm.at[c, pl.ds(s * NPS, NPS)])


def _tc_norms_body(dega_ref, degb_ref, out_ref):
    deg_i = dega_ref[0, :N, :1] + dega_ref[1, :N, :1]
    deg_o = degb_ref[0, :N, 64:65] + degb_ref[1, :N, 64:65]
    ns = lax.rsqrt(jnp.maximum(deg_o, 1.0))
    nd = lax.rsqrt(jnp.maximum(deg_i, 1.0))
    out_ref[...] = jnp.concatenate(
        [ns, nd, jnp.zeros((N, 6), jnp.float32)], axis=1)


def _tc_pre_body(feat_ref, nrm_ref, w_ref, out_ref):
    ns = nrm_ref[:, 0:1]
    out_ref[:N, :] = jnp.dot(feat_ref[...] * ns, w_ref[...],
                             preferred_element_type=jnp.float32)
    out_ref[N:, :] = jnp.zeros((NPAD, D), jnp.float32)


def _tc_mid_body(aggp_ref, nrm_ref, b_ref, w_ref, out_ref):
    ns = nrm_ref[:, 0:1]
    nd = nrm_ref[:, 1:2]
    agg = aggp_ref[0, :N] + aggp_ref[1, :N]
    x = jnp.maximum(agg * nd + b_ref[...], 0.0)
    out_ref[:N, :] = jnp.dot(x * ns, w_ref[...],
                             preferred_element_type=jnp.float32)
    out_ref[N:, :] = jnp.zeros((NPAD, D), jnp.float32)


def _tc_fin_body(aggp_ref, nrm_ref, b_ref, wf_ref, bf_ref, out_ref):
    nd = nrm_ref[:, 1:2]
    agg = aggp_ref[0, :N] + aggp_ref[1, :N]
    x = jnp.maximum(agg * nd + b_ref[...], 0.0)
    ssum = jnp.sum(x, axis=0, keepdims=True)
    out_ref[...] = (jnp.dot(ssum, wf_ref[...],
                            preferred_element_type=jnp.float32)
                    * (1.0 / N) + bf_ref[...])


_tc_norms = pl.pallas_call(
    _tc_norms_body, out_shape=jax.ShapeDtypeStruct((N, 8), jnp.float32))
_tc_pre = pl.pallas_call(
    _tc_pre_body, out_shape=jax.ShapeDtypeStruct((NH, D), jnp.float32))
_tc_mid = pl.pallas_call(
    _tc_mid_body, out_shape=jax.ShapeDtypeStruct((NH, D), jnp.float32))
_tc_fin = pl.pallas_call(
    _tc_fin_body, out_shape=jax.ShapeDtypeStruct((1, DOUT), jnp.float32))


def kernel(features, edge_index, W1, b1, W2, b2, W3, b3, Wf, bf):
    pad_idx = (N + jnp.arange(EP - E, dtype=jnp.int32) % NPAD)
    src2d = jnp.concatenate([edge_index[0], pad_idx]).reshape(EP_ROWS, LW)
    dst2d = jnp.concatenate([edge_index[1], pad_idx]).reshape(EP_ROWS, LW)
    zerosD = jnp.zeros((NPS, D), jnp.float32)
    t_lo = jnp.concatenate(
        [jnp.ones((NH, 64), jnp.float32), jnp.zeros((NH, 64), jnp.float32)], 1)
    t_hi = jnp.concatenate(
        [jnp.zeros((NH, 64), jnp.float32), jnp.ones((NH, 64), jnp.float32)], 1)

    dega = _sc_aggregate(t_lo, src2d, dst2d, zerosD)
    degb = _sc_aggregate(t_hi, dst2d, src2d, zerosD)
    nrm = _tc_norms(dega, degb)
    h = _tc_pre(features, nrm, W1)
    aggp = _sc_aggregate(h, src2d, dst2d, zerosD)
    h = _tc_mid(aggp, nrm, b1.reshape(1, D), W2)
    aggp = _sc_aggregate(h, src2d, dst2d, zerosD)
    h = _tc_mid(aggp, nrm, b2.reshape(1, D), W3)
    aggp = _sc_aggregate(h, src2d, dst2d, zerosD)
    out = _tc_fin(aggp, nrm, b3.reshape(1, D), Wf, bf.reshape(1, DOUT))
    return out.reshape(DOUT)


# speedup vs baseline: 8.4914x; 1.5124x over previous
import functools

import jax
import jax.numpy as jnp
from jax import lax
from jax.experimental import pallas as pl
from jax.experimental.pallas import tpu as pltpu
from jax.experimental.pallas import tpu_sc as plsc

N = 10000
E = 320000
D = 128
DOUT = 64

NC = 2
NS = 16
LW = 128
RPT = 80
EP_ROWS = NC * NS * RPT
EP = EP_ROWS * LW
NH = 10240
NPS = NH // NS
NPAD = NH - N

_mesh = plsc.VectorSubcoreMesh(
    core_axis_name="c", subcore_axis_name="s", num_cores=NC, num_subcores=NS
)


@functools.partial(
    pl.kernel,
    out_type=jax.ShapeDtypeStruct((NC, NH, D), jnp.float32),
    mesh=_mesh,
    scratch_types=[
        pltpu.VMEM((RPT // 2, LW), jnp.int32),
        pltpu.VMEM((RPT // 2, LW), jnp.int32),
        pltpu.VMEM((LW, D), jnp.float32),
        pltpu.VMEM((LW, D), jnp.float32),
        pltpu.VMEM_SHARED((NH, D), jnp.float32),
        pltpu.SemaphoreType.DMA,
        pltpu.SemaphoreType.DMA,
    ],
)
def _sc_aggregate(h_hbm, src2d, dst2d, zeros_hbm, out_hbm,
                  idx_s, idx_d, rows0, rows1, acc, sem0, sem1):
    c = lax.axis_index("c")
    s = lax.axis_index("s")
    hrpt = RPT // 2
    pltpu.sync_copy(zeros_hbm, acc.at[pl.ds(s * NPS, NPS)])
    base = (c * NS + s) * RPT
    plsc.subcore_barrier()

    bufs = ((rows0, sem0), (rows1, sem1))
    for half in range(2):
        pltpu.sync_copy(src2d.at[pl.ds(base + half * hrpt, hrpt)], idx_s)
        pltpu.sync_copy(dst2d.at[pl.ds(base + half * hrpt, hrpt)], idx_d)
        pltpu.async_copy(h_hbm.at[idx_s.at[0]], rows0, sem0)
        pltpu.async_copy(h_hbm.at[idx_s.at[1]], rows1, sem1)

        def body(k, carry):
            for b in range(2):
                i = 2 * k + b
                rows, sem = bufs[b]
                pltpu.make_async_copy(h_hbm.at[idx_s.at[i]], rows, sem).wait()
                pltpu.sync_copy(rows, acc.at[idx_d.at[i]], add=True)
                pltpu.async_copy(h_hbm.at[idx_s.at[i + 2]], rows, sem)
            return carry

        lax.fori_loop(0, hrpt // 2 - 1, body, 0)
        for b in range(2):
            i = hrpt - 2 + b
            rows, sem = bufs[b]
            pltpu.make_async_copy(h_hbm.at[idx_s.at[i]], rows, sem).wait()
            pltpu.sync_copy(rows, acc.at[idx_d.at[i]], add=True)

    plsc.subcore_barrier()
    pltpu.sync_copy(acc.at[pl.ds(s * NPS, NPS)],
                    out_hbm.at[c, pl.ds(s * NPS, NPS)])


def _tc_norms_body(dega_ref, degb_ref, out_ref):
    deg_i = dega_ref[0, :N, :1] + dega_ref[1, :N, :1]
    deg_o = degb_ref[0, :N, 64:65] + degb_ref[1, :N, 64:65]
    ns = lax.rsqrt(jnp.maximum(deg_o, 1.0))
    nd = lax.rsqrt(jnp.maximum(deg_i, 1.0))
    out_ref[...] = jnp.concatenate(
        [ns, nd, jnp.zeros((N, 6), jnp.float32)], axis=1)


def _tc_pre_body(feat_ref, nrm_ref, w_ref, out_ref):
    ns = nrm_ref[:, 0:1]
    out_ref[:N, :] = jnp.dot(feat_ref[...] * ns, w_ref[...],
                             preferred_element_type=jnp.float32)
    out_ref[N:, :] = jnp.zeros((NPAD, D), jnp.float32)


def _tc_mid_body(aggp_ref, nrm_ref, b_ref, w_ref, out_ref):
    ns = nrm_ref[:, 0:1]
    nd = nrm_ref[:, 1:2]
    agg = aggp_ref[0, :N] + aggp_ref[1, :N]
    x = jnp.maximum(agg * nd + b_ref[...], 0.0)
    out_ref[:N, :] = jnp.dot(x * ns, w_ref[...],
                             preferred_element_type=jnp.float32)
    out_ref[N:, :] = jnp.zeros((NPAD, D), jnp.float32)


def _tc_fin_body(aggp_ref, nrm_ref, b_ref, wf_ref, bf_ref, out_ref):
    nd = nrm_ref[:, 1:2]
    agg = aggp_ref[0, :N] + aggp_ref[1, :N]
    x = jnp.maximum(agg * nd + b_ref[...], 0.0)
    ssum = jnp.sum(x, axis=0, keepdims=True)
    out_ref[...] = (jnp.dot(ssum, wf_ref[...],
                            preferred_element_type=jnp.float32)
                    * (1.0 / N) + bf_ref[...])


_tc_norms = pl.pallas_call(
    _tc_norms_body, out_shape=jax.ShapeDtypeStruct((N, 8), jnp.float32))
_tc_pre = pl.pallas_call(
    _tc_pre_body, out_shape=jax.ShapeDtypeStruct((NH, D), jnp.float32))
_tc_mid = pl.pallas_call(
    _tc_mid_body, out_shape=jax.ShapeDtypeStruct((NH, D), jnp.float32))
_tc_fin = pl.pallas_call(
    _tc_fin_body, out_shape=jax.ShapeDtypeStruct((1, DOUT), jnp.float32))


def kernel(features, edge_index, W1, b1, W2, b2, W3, b3, Wf, bf):
    pad_idx = (N + jnp.arange(EP - E, dtype=jnp.int32) % NPAD)
    src2d = jnp.concatenate([edge_index[0], pad_idx]).reshape(EP_ROWS, LW)
    dst2d = jnp.concatenate([edge_index[1], pad_idx]).reshape(EP_ROWS, LW)
    zerosD = jnp.zeros((NPS, D), jnp.float32)
    t_lo = jnp.concatenate(
        [jnp.ones((NH, 64), jnp.float32), jnp.zeros((NH, 64), jnp.float32)], 1)
    t_hi = jnp.concatenate(
        [jnp.zeros((NH, 64), jnp.float32), jnp.ones((NH, 64), jnp.float32)], 1)

    dega = _sc_aggregate(t_lo, src2d, dst2d, zerosD)
    degb = _sc_aggregate(t_hi, dst2d, src2d, zerosD)
    nrm = _tc_norms(dega, degb)
    h = _tc_pre(features, nrm, W1)
    aggp = _sc_aggregate(h, src2d, dst2d, zerosD)
    h = _tc_mid(aggp, nrm, b1.reshape(1, D), W2)
    aggp = _sc_aggregate(h, src2d, dst2d, zerosD)
    h = _tc_mid(aggp, nrm, b2.reshape(1, D), W3)
    aggp = _sc_aggregate(h, src2d, dst2d, zerosD)
    out = _tc_fin(aggp, nrm, b3.reshape(1, D), Wf, bf.reshape(1, DOUT))
    return out.reshape(DOUT)


# speedup vs baseline: 8.5760x; 1.0100x over previous
import functools

import jax
import jax.numpy as jnp
from jax import lax
from jax.experimental import pallas as pl
from jax.experimental.pallas import tpu as pltpu
from jax.experimental.pallas import tpu_sc as plsc

N = 10000
E = 320000
D = 128
DOUT = 64

NC = 2
NS = 16
LW = 128
RPT = 80
EP_ROWS = NC * NS * RPT
EP = EP_ROWS * LW
NH = 10240
NPS = NH // NS
NPAD = NH - N

_mesh = plsc.VectorSubcoreMesh(
    core_axis_name="c", subcore_axis_name="s", num_cores=NC, num_subcores=NS
)


@functools.partial(
    pl.kernel,
    out_type=jax.ShapeDtypeStruct((NC, NH, D), jnp.float32),
    mesh=_mesh,
    scratch_types=[
        pltpu.VMEM((RPT // 2, LW), jnp.int32),
        pltpu.VMEM((RPT // 2, LW), jnp.int32),
        pltpu.VMEM((LW, D), jnp.float32),
        pltpu.VMEM((LW, D), jnp.float32),
        pltpu.VMEM_SHARED((NH, D), jnp.float32),
        pltpu.SemaphoreType.DMA,
        pltpu.SemaphoreType.DMA,
    ],
)
def _sc_aggregate(h_hbm, src2d, dst2d, zeros_hbm, out_hbm,
                  idx_s, idx_d, rows0, rows1, acc, sem0, sem1):
    c = lax.axis_index("c")
    s = lax.axis_index("s")
    hrpt = RPT // 2
    zcp = pltpu.async_copy(zeros_hbm, acc.at[pl.ds(s * NPS, NPS)], sem0)
    base = (c * NS + s) * RPT
    pltpu.sync_copy(src2d.at[pl.ds(base, hrpt)], idx_s)
    pltpu.sync_copy(dst2d.at[pl.ds(base, hrpt)], idx_d)
    zcp.wait()
    plsc.subcore_barrier()

    bufs = ((rows0, sem0), (rows1, sem1))
    for half in range(2):
        if half:
            pltpu.sync_copy(src2d.at[pl.ds(base + half * hrpt, hrpt)], idx_s)
            pltpu.sync_copy(dst2d.at[pl.ds(base + half * hrpt, hrpt)], idx_d)
        pltpu.async_copy(h_hbm.at[idx_s.at[0]], rows0, sem0)
        pltpu.async_copy(h_hbm.at[idx_s.at[1]], rows1, sem1)

        def body(k, carry):
            for b in range(2):
                i = 2 * k + b
                rows, sem = bufs[b]
                pltpu.make_async_copy(h_hbm.at[idx_s.at[i]], rows, sem).wait()
                pltpu.sync_copy(rows, acc.at[idx_d.at[i]], add=True)
                pltpu.async_copy(h_hbm.at[idx_s.at[i + 2]], rows, sem)
            return carry

        lax.fori_loop(0, hrpt // 2 - 1, body, 0)
        for b in range(2):
            i = hrpt - 2 + b
            rows, sem = bufs[b]
            pltpu.make_async_copy(h_hbm.at[idx_s.at[i]], rows, sem).wait()
            pltpu.sync_copy(rows, acc.at[idx_d.at[i]], add=True)

    plsc.subcore_barrier()
    pltpu.sync_copy(acc.at[pl.ds(s * NPS, NPS)],
                    out_hbm.at[c, pl.ds(s * NPS, NPS)])


def _tc_norms_body(dega_ref, degb_ref, out_ref):
    deg_i = dega_ref[0, :N, :1] + dega_ref[1, :N, :1]
    deg_o = degb_ref[0, :N, 64:65] + degb_ref[1, :N, 64:65]
    ns = lax.rsqrt(jnp.maximum(deg_o, 1.0))
    nd = lax.rsqrt(jnp.maximum(deg_i, 1.0))
    out_ref[...] = jnp.concatenate(
        [ns, nd, jnp.zeros((N, 6), jnp.float32)], axis=1)


def _tc_pre_body(feat_ref, nrm_ref, w_ref, out_ref):
    ns = nrm_ref[:, 0:1]
    out_ref[:N, :] = jnp.dot(feat_ref[...] * ns, w_ref[...],
                             preferred_element_type=jnp.float32)
    out_ref[N:, :] = jnp.zeros((NPAD, D), jnp.float32)


def _tc_mid_body(aggp_ref, nrm_ref, b_ref, w_ref, out_ref):
    ns = nrm_ref[:, 0:1]
    nd = nrm_ref[:, 1:2]
    agg = aggp_ref[0, :N] + aggp_ref[1, :N]
    x = jnp.maximum(agg * nd + b_ref[...], 0.0)
    out_ref[:N, :] = jnp.dot(x * ns, w_ref[...],
                             preferred_element_type=jnp.float32)
    out_ref[N:, :] = jnp.zeros((NPAD, D), jnp.float32)


def _tc_fin_body(aggp_ref, nrm_ref, b_ref, wf_ref, bf_ref, out_ref):
    nd = nrm_ref[:, 1:2]
    agg = aggp_ref[0, :N] + aggp_ref[1, :N]
    x = jnp.maximum(agg * nd + b_ref[...], 0.0)
    ssum = jnp.sum(x, axis=0, keepdims=True)
    out_ref[...] = (jnp.dot(ssum, wf_ref[...],
                            preferred_element_type=jnp.float32)
                    * (1.0 / N) + bf_ref[...])


_tc_norms = pl.pallas_call(
    _tc_norms_body, out_shape=jax.ShapeDtypeStruct((N, 8), jnp.float32))
_tc_pre = pl.pallas_call(
    _tc_pre_body, out_shape=jax.ShapeDtypeStruct((NH, D), jnp.float32))
_tc_mid = pl.pallas_call(
    _tc_mid_body, out_shape=jax.ShapeDtypeStruct((NH, D), jnp.float32))
_tc_fin = pl.pallas_call(
    _tc_fin_body, out_shape=jax.ShapeDtypeStruct((1, DOUT), jnp.float32))


def kernel(features, edge_index, W1, b1, W2, b2, W3, b3, Wf, bf):
    pad_idx = (N + jnp.arange(EP - E, dtype=jnp.int32) % NPAD)
    src2d = jnp.concatenate([edge_index[0], pad_idx]).reshape(EP_ROWS, LW)
    dst2d = jnp.concatenate([edge_index[1], pad_idx]).reshape(EP_ROWS, LW)
    zerosD = jnp.zeros((NPS, D), jnp.float32)
    t_lo = jnp.concatenate(
        [jnp.ones((NH, 64), jnp.float32), jnp.zeros((NH, 64), jnp.float32)], 1)
    t_hi = jnp.concatenate(
        [jnp.zeros((NH, 64), jnp.float32), jnp.ones((NH, 64), jnp.float32)], 1)

    dega = _sc_aggregate(t_lo, src2d, dst2d, zerosD)
    degb = _sc_aggregate(t_hi, dst2d, src2d, zerosD)
    nrm = _tc_norms(dega, degb)
    h = _tc_pre(features, nrm, W1)
    aggp = _sc_aggregate(h, src2d, dst2d, zerosD)
    h = _tc_mid(aggp, nrm, b1.reshape(1, D), W2)
    aggp = _sc_aggregate(h, src2d, dst2d, zerosD)
    h = _tc_mid(aggp, nrm, b2.reshape(1, D), W3)
    aggp = _sc_aggregate(h, src2d, dst2d, zerosD)
    out = _tc_fin(aggp, nrm, b3.reshape(1, D), Wf, bf.reshape(1, DOUT))
    return out.reshape(DOUT)


# speedup vs baseline: 8.6655x; 1.0104x over previous
import functools

import jax
import jax.numpy as jnp
from jax import lax
from jax.experimental import pallas as pl
from jax.experimental.pallas import tpu as pltpu
from jax.experimental.pallas import tpu_sc as plsc

N = 10000
E = 320000
D = 128
DOUT = 64

NC = 2
NS = 16
LW = 128
RPT = 80
EP_ROWS = NC * NS * RPT
EP = EP_ROWS * LW
NH = 10240
NPS = NH // NS
NPAD = NH - N

_mesh = plsc.VectorSubcoreMesh(
    core_axis_name="c", subcore_axis_name="s", num_cores=NC, num_subcores=NS
)


@functools.partial(
    pl.kernel,
    out_type=jax.ShapeDtypeStruct((NC, NH, D), jnp.float32),
    mesh=_mesh,
    scratch_types=[
        pltpu.VMEM((RPT // 2, LW), jnp.int32),
        pltpu.VMEM((RPT // 2, LW), jnp.int32),
        pltpu.VMEM((LW, D), jnp.float32),
        pltpu.VMEM((LW, D), jnp.float32),
        pltpu.VMEM_SHARED((NH, D), jnp.float32),
        pltpu.SemaphoreType.DMA,
        pltpu.SemaphoreType.DMA,
    ],
)
def _sc_aggregate(h_hbm, src2d, dst2d, zeros_hbm, out_hbm,
                  idx_s, idx_d, rows0, rows1, acc, sem0, sem1):
    c = lax.axis_index("c")
    s = lax.axis_index("s")
    hrpt = RPT // 2
    zcp = pltpu.async_copy(zeros_hbm, acc.at[pl.ds(s * NPS, NPS)], sem0)
    base = (c * NS + s) * RPT
    pltpu.sync_copy(src2d.at[pl.ds(base, hrpt)], idx_s)
    pltpu.sync_copy(dst2d.at[pl.ds(base, hrpt)], idx_d)
    zcp.wait()
    plsc.subcore_barrier()

    bufs = ((rows0, sem0), (rows1, sem1))
    for half in range(2):
        if half:
            pltpu.sync_copy(src2d.at[pl.ds(base + half * hrpt, hrpt)], idx_s)
            pltpu.sync_copy(dst2d.at[pl.ds(base + half * hrpt, hrpt)], idx_d)
        pltpu.async_copy(h_hbm.at[idx_s.at[0]], rows0, sem0)
        pltpu.async_copy(h_hbm.at[idx_s.at[1]], rows1, sem1)

        def body(k, carry):
            for b in range(2):
                i = 2 * k + b
                rows, sem = bufs[b]
                pltpu.make_async_copy(h_hbm.at[idx_s.at[i]], rows, sem).wait()
                pltpu.sync_copy(rows, acc.at[idx_d.at[i]], add=True)
                pltpu.async_copy(h_hbm.at[idx_s.at[i + 2]], rows, sem)
            return carry

        lax.fori_loop(0, hrpt // 2 - 1, body, 0)
        for b in range(2):
            i = hrpt - 2 + b
            rows, sem = bufs[b]
            pltpu.make_async_copy(h_hbm.at[idx_s.at[i]], rows, sem).wait()
            pltpu.sync_copy(rows, acc.at[idx_d.at[i]], add=True)

    plsc.subcore_barrier()
    pltpu.sync_copy(acc.at[pl.ds(s * NPS, NPS)],
                    out_hbm.at[c, pl.ds(s * NPS, NPS)])


def _tc_pre_body(feat_ref, dega_ref, degb_ref, w_ref, out_ref, nrm_ref):
    deg_i = dega_ref[0, :N, :1] + dega_ref[1, :N, :1]
    deg_o = degb_ref[0, :N, 64:65] + degb_ref[1, :N, 64:65]
    ns = lax.rsqrt(jnp.maximum(deg_o, 1.0))
    nd = lax.rsqrt(jnp.maximum(deg_i, 1.0))
    nrm_ref[...] = jnp.concatenate(
        [ns, nd, jnp.zeros((N, 6), jnp.float32)], axis=1)
    out_ref[:N, :] = jnp.dot(feat_ref[...] * ns, w_ref[...],
                             preferred_element_type=jnp.float32)
    out_ref[N:, :] = jnp.zeros((NPAD, D), jnp.float32)


def _tc_mid_body(aggp_ref, nrm_ref, b_ref, w_ref, out_ref):
    ns = nrm_ref[:, 0:1]
    nd = nrm_ref[:, 1:2]
    agg = aggp_ref[0, :N] + aggp_ref[1, :N]
    x = jnp.maximum(agg * nd + b_ref[...], 0.0)
    out_ref[:N, :] = jnp.dot(x * ns, w_ref[...],
                             preferred_element_type=jnp.float32)
    out_ref[N:, :] = jnp.zeros((NPAD, D), jnp.float32)


def _tc_fin_body(aggp_ref, nrm_ref, b_ref, wf_ref, bf_ref, out_ref):
    nd = nrm_ref[:, 1:2]
    agg = aggp_ref[0, :N] + aggp_ref[1, :N]
    x = jnp.maximum(agg * nd + b_ref[...], 0.0)
    ssum = jnp.sum(x, axis=0, keepdims=True)
    out_ref[...] = (jnp.dot(ssum, wf_ref[...],
                            preferred_element_type=jnp.float32)
                    * (1.0 / N) + bf_ref[...])


_tc_pre = pl.pallas_call(
    _tc_pre_body, out_shape=(jax.ShapeDtypeStruct((NH, D), jnp.float32),
                             jax.ShapeDtypeStruct((N, 8), jnp.float32)))
_tc_mid = pl.pallas_call(
    _tc_mid_body, out_shape=jax.ShapeDtypeStruct((NH, D), jnp.float32))
_tc_fin = pl.pallas_call(
    _tc_fin_body, out_shape=jax.ShapeDtypeStruct((1, DOUT), jnp.float32))


def kernel(features, edge_index, W1, b1, W2, b2, W3, b3, Wf, bf):
    pad_idx = (N + jnp.arange(EP - E, dtype=jnp.int32) % NPAD)
    src2d = jnp.concatenate([edge_index[0], pad_idx]).reshape(EP_ROWS, LW)
    dst2d = jnp.concatenate([edge_index[1], pad_idx]).reshape(EP_ROWS, LW)
    zerosD = jnp.zeros((NPS, D), jnp.float32)
    t_lo = jnp.concatenate(
        [jnp.ones((NH, 64), jnp.float32), jnp.zeros((NH, 64), jnp.float32)], 1)
    t_hi = jnp.concatenate(
        [jnp.zeros((NH, 64), jnp.float32), jnp.ones((NH, 64), jnp.float32)], 1)

    dega = _sc_aggregate(t_lo, src2d, dst2d, zerosD)
    degb = _sc_aggregate(t_hi, dst2d, src2d, zerosD)
    h, nrm = _tc_pre(features, dega, degb, W1)
    aggp = _sc_aggregate(h, src2d, dst2d, zerosD)
    h = _tc_mid(aggp, nrm, b1.reshape(1, D), W2)
    aggp = _sc_aggregate(h, src2d, dst2d, zerosD)
    h = _tc_mid(aggp, nrm, b2.reshape(1, D), W3)
    aggp = _sc_aggregate(h, src2d, dst2d, zerosD)
    out = _tc_fin(aggp, nrm, b3.reshape(1, D), Wf, bf.reshape(1, DOUT))
    return out.reshape(DOUT)
